# Initial kernel scaffold; baseline (speedup 1.0000x reference)
import jax
import jax.numpy as jnp
from jax.experimental import pallas as pl


def kernel(x, adj, W1, a1_src, a1_dst, W2, a2_src, a2_dst, Wd, bd):
    raise NotImplementedError("write your pallas kernel here")


# speedup vs baseline: 1.5932x; 1.5932x over previous
import functools

import jax
import jax.numpy as jnp
from jax.experimental import pallas as pl


def _proj_body(x_ref, w_ref, h_ref):
    h_ref[0] = jax.lax.dot_general(
        x_ref[...], w_ref[0],
        dimension_numbers=(((1,), (0,)), ((), ())),
        preferred_element_type=jnp.float32)


def _proj(x, w):
    heads, fin, fout = w.shape
    n = x.shape[0]
    return pl.pallas_call(
        _proj_body,
        grid=(heads,),
        in_specs=[
            pl.BlockSpec((n, fin), lambda h: (0, 0)),
            pl.BlockSpec((1, fin, fout), lambda h: (h, 0, 0)),
        ],
        out_specs=pl.BlockSpec((1, n, fout), lambda h: (h, 0, 0)),
        out_shape=jax.ShapeDtypeStruct((heads, n, fout), jnp.float32),
    )(x, w)


def _attn_body(adj_ref, hall_ref, asrc_ref, adst_ref, out_ref, *, blk):
    i = pl.program_id(0)
    h = pl.program_id(1)
    n = adj_ref.shape[1]

    adj_blk = adj_ref[...]
    rows = jax.lax.broadcasted_iota(jnp.int32, (blk, n), 0) + i * blk
    cols = jax.lax.broadcasted_iota(jnp.int32, (blk, n), 1)
    mask = (adj_blk > 0.99) | (rows == cols)

    hh = hall_ref[h]
    a_src = asrc_ref[h]
    a_dst = adst_ref[h]
    hh_blk = hall_ref[h, pl.ds(i * blk, blk), :]
    es = jax.lax.dot_general(hh_blk, a_src, (((1,), (1,)), ((), ())),
                             preferred_element_type=jnp.float32)
    ed = jax.lax.dot_general(a_dst, hh, (((1,), (1,)), ((), ())),
                             preferred_element_type=jnp.float32)
    e = es + ed
    e = jnp.where(e >= 0, e, 0.2 * e)
    e = jnp.where(mask, e, -1e9)
    m = jnp.max(e, axis=1, keepdims=True)
    p = jnp.exp(e - m)
    denom = jnp.sum(p, axis=1, keepdims=True)
    alpha = p / denom
    o = jax.lax.dot_general(alpha, hh, (((1,), (0,)), ((), ())),
                            preferred_element_type=jnp.float32)
    out_ref[0] = jnp.where(o > 0, o, jnp.exp(o) - 1.0)


def _attn(adj, hall, a_src, a_dst, blk):
    heads, n, f = hall.shape
    return pl.pallas_call(
        functools.partial(_attn_body, blk=blk),
        grid=(n // blk, heads),
        in_specs=[
            pl.BlockSpec((blk, n), lambda i, h: (i, 0)),
            pl.BlockSpec((heads, n, f), lambda i, h: (0, 0, 0)),
            pl.BlockSpec((heads, 1, f), lambda i, h: (0, 0, 0)),
            pl.BlockSpec((heads, 1, f), lambda i, h: (0, 0, 0)),
        ],
        out_specs=pl.BlockSpec((1, blk, f), lambda i, h: (h, i, 0)),
        out_shape=jax.ShapeDtypeStruct((heads, n, f), jnp.float32),
    )(adj, hall, a_src.reshape(heads, 1, f), a_dst.reshape(heads, 1, f))


def _head_body(h2_ref, wd_ref, bd_ref, out_ref):
    s = jnp.sum(h2_ref[...], axis=0, keepdims=True)
    ss = jnp.sum(s * s, axis=1, keepdims=True)
    norm = jnp.maximum(jnp.sqrt(ss), 1e-12)
    s = s / norm
    out_ref[...] = jax.lax.dot_general(
        s, wd_ref[...], (((1,), (0,)), ((), ())),
        preferred_element_type=jnp.float32) + bd_ref[...]


def _head(h2, wd, bd):
    n, d = h2.shape
    return pl.pallas_call(
        _head_body,
        grid=(1,),
        in_specs=[
            pl.BlockSpec((n, d), lambda i: (0, 0)),
            pl.BlockSpec((d, 1), lambda i: (0, 0)),
            pl.BlockSpec((1, 1), lambda i: (0, 0)),
        ],
        out_specs=pl.BlockSpec((1, 1), lambda i: (0, 0)),
        out_shape=jax.ShapeDtypeStruct((1, 1), jnp.float32),
    )(h2, wd, bd)


def kernel(x, adj, W1, a1_src, a1_dst, W2, a2_src, a2_dst, Wd, bd):
    n = x.shape[0]
    blk = 256 if n % 256 == 0 else n

    h1p = _proj(x, W1)
    h1 = _attn(adj, h1p, a1_src, a1_dst, blk)
    h1 = jnp.transpose(h1, (1, 0, 2)).reshape(n, -1)
    h2p = _proj(h1, W2)
    h2 = _attn(adj, h2p, a2_src, a2_dst, blk)
    h2 = jnp.transpose(h2, (1, 0, 2)).reshape(n, -1)
    out = _head(h2, Wd, bd.reshape(1, 1))
    return out.reshape(1)


# speedup vs baseline: 1.7815x; 1.1182x over previous
import functools

import jax
import jax.numpy as jnp
from jax.experimental import pallas as pl
from jax.experimental.pallas import tpu as pltpu


def _proj_body(x_ref, w_ref, h_ref):
    h_ref[0] = jax.lax.dot_general(
        x_ref[...], w_ref[0],
        dimension_numbers=(((1,), (0,)), ((), ())),
        preferred_element_type=jnp.float32)


def _proj(x, w):
    heads, fin, fout = w.shape
    n = x.shape[0]
    return pl.pallas_call(
        _proj_body,
        grid=(heads,),
        in_specs=[
            pl.BlockSpec((n, fin), lambda h: (0, 0)),
            pl.BlockSpec((1, fin, fout), lambda h: (h, 0, 0)),
        ],
        out_specs=pl.BlockSpec((1, n, fout), lambda h: (h, 0, 0)),
        out_shape=jax.ShapeDtypeStruct((heads, n, fout), jnp.float32),
    )(x, w)


def _attn_common(hall_ref, asrc_ref, adst_ref, out_ref, bias_ref, blk):
    i = pl.program_id(0)
    h = pl.program_id(1)
    hh = hall_ref[h]
    hh_blk = hall_ref[h, pl.ds(i * blk, blk), :]
    es = jax.lax.dot_general(hh_blk, asrc_ref[h], (((1,), (1,)), ((), ())),
                             preferred_element_type=jnp.float32)
    ed = jax.lax.dot_general(adst_ref[h], hh, (((1,), (1,)), ((), ())),
                             preferred_element_type=jnp.float32)
    e = es + ed
    e = jnp.maximum(e, 0.2 * e) + bias_ref[...]
    m = jnp.max(e, axis=1, keepdims=True)
    p = jnp.exp(e - m)
    denom = jnp.sum(p, axis=1, keepdims=True)
    o = jax.lax.dot_general(p, hh, (((1,), (0,)), ((), ())),
                            preferred_element_type=jnp.float32) / denom
    out_ref[0] = jnp.where(o > 0, o, jnp.exp(o) - 1.0)


def _attn1_body(adj_ref, hall_ref, asrc_ref, adst_ref, out_ref, mask_ref,
                bias_ref, *, blk):
    i = pl.program_id(0)
    h = pl.program_id(1)
    n = adj_ref.shape[1]

    @pl.when(h == 0)
    def _():
        rows = jax.lax.broadcasted_iota(jnp.int32, (blk, n), 0) + i * blk
        cols = jax.lax.broadcasted_iota(jnp.int32, (blk, n), 1)
        msk = (adj_ref[...] > 0.99) | (rows == cols)
        mask_ref[...] = msk.astype(jnp.int8)
        bias_ref[...] = jnp.where(msk, 0.0, -1e9)

    _attn_common(hall_ref, asrc_ref, adst_ref, out_ref, bias_ref, blk)


def _attn2_body(mask_ref, hall_ref, asrc_ref, adst_ref, out_ref,
                bias_ref, *, blk):
    h = pl.program_id(1)

    @pl.when(h == 0)
    def _():
        bias_ref[...] = mask_ref[...].astype(jnp.float32) * 1e9 - 1e9

    _attn_common(hall_ref, asrc_ref, adst_ref, out_ref, bias_ref, blk)


def _attn1(adj, hall, a_src, a_dst, blk):
    heads, n, f = hall.shape
    return pl.pallas_call(
        functools.partial(_attn1_body, blk=blk),
        grid=(n // blk, heads),
        in_specs=[
            pl.BlockSpec((blk, n), lambda i, h: (i, 0)),
            pl.BlockSpec((heads, n, f), lambda i, h: (0, 0, 0)),
            pl.BlockSpec((heads, 1, f), lambda i, h: (0, 0, 0)),
            pl.BlockSpec((heads, 1, f), lambda i, h: (0, 0, 0)),
        ],
        out_specs=[
            pl.BlockSpec((1, blk, f), lambda i, h: (h, i, 0)),
            pl.BlockSpec((blk, n), lambda i, h: (i, 0)),
        ],
        out_shape=[
            jax.ShapeDtypeStruct((heads, n, f), jnp.float32),
            jax.ShapeDtypeStruct((n, n), jnp.int8),
        ],
        scratch_shapes=[pltpu.VMEM((blk, n), jnp.float32)],
    )(adj, hall, a_src.reshape(heads, 1, f), a_dst.reshape(heads, 1, f))


def _attn2(mask, hall, a_src, a_dst, blk):
    heads, n, f = hall.shape
    return pl.pallas_call(
        functools.partial(_attn2_body, blk=blk),
        grid=(n // blk, heads),
        in_specs=[
            pl.BlockSpec((blk, n), lambda i, h: (i, 0)),
            pl.BlockSpec((heads, n, f), lambda i, h: (0, 0, 0)),
            pl.BlockSpec((heads, 1, f), lambda i, h: (0, 0, 0)),
            pl.BlockSpec((heads, 1, f), lambda i, h: (0, 0, 0)),
        ],
        out_specs=pl.BlockSpec((1, blk, f), lambda i, h: (h, i, 0)),
        out_shape=jax.ShapeDtypeStruct((heads, n, f), jnp.float32),
        scratch_shapes=[pltpu.VMEM((blk, n), jnp.float32)],
    )(mask, hall, a_src.reshape(heads, 1, f), a_dst.reshape(heads, 1, f))


def _head_body(h2_ref, wd_ref, bd_ref, out_ref):
    s = jnp.sum(h2_ref[...], axis=0, keepdims=True)
    ss = jnp.sum(s * s, axis=1, keepdims=True)
    norm = jnp.maximum(jnp.sqrt(ss), 1e-12)
    s = s / norm
    out_ref[...] = jax.lax.dot_general(
        s, wd_ref[...], (((1,), (0,)), ((), ())),
        preferred_element_type=jnp.float32) + bd_ref[...]


def _head(h2, wd, bd):
    n, d = h2.shape
    return pl.pallas_call(
        _head_body,
        grid=(1,),
        in_specs=[
            pl.BlockSpec((n, d), lambda i: (0, 0)),
            pl.BlockSpec((d, 1), lambda i: (0, 0)),
            pl.BlockSpec((1, 1), lambda i: (0, 0)),
        ],
        out_specs=pl.BlockSpec((1, 1), lambda i: (0, 0)),
        out_shape=jax.ShapeDtypeStruct((1, 1), jnp.float32),
    )(h2, wd, bd)


def kernel(x, adj, W1, a1_src, a1_dst, W2, a2_src, a2_dst, Wd, bd):
    n = x.shape[0]
    blk = 256 if n % 256 == 0 else n

    h1p = _proj(x, W1)
    h1, mask = _attn1(adj, h1p, a1_src, a1_dst, blk)
    h1 = jnp.transpose(h1, (1, 0, 2)).reshape(n, -1)
    h2p = _proj(h1, W2)
    h2 = _attn2(mask, h2p, a2_src, a2_dst, blk)
    h2 = jnp.transpose(h2, (1, 0, 2)).reshape(n, -1)
    out = _head(h2, Wd, bd.reshape(1, 1))
    return out.reshape(1)


# speedup vs baseline: 2.1741x; 1.2204x over previous
import functools

import jax
import jax.numpy as jnp
from jax.experimental import pallas as pl
from jax.experimental.pallas import tpu as pltpu


def _proj_body(x_ref, w_ref, h_ref):
    f = w_ref.shape[2]
    n = x_ref.shape[0]
    h_ref[0, :, :f] = jax.lax.dot_general(
        x_ref[...], w_ref[0],
        dimension_numbers=(((1,), (0,)), ((), ())),
        preferred_element_type=jnp.float32)
    h_ref[0, :, f:] = jnp.ones((n, 1), jnp.float32)


def _proj(x, w):
    heads, fin, fout = w.shape
    n = x.shape[0]
    return pl.pallas_call(
        _proj_body,
        grid=(heads,),
        in_specs=[
            pl.BlockSpec((n, fin), lambda h: (0, 0)),
            pl.BlockSpec((1, fin, fout), lambda h: (h, 0, 0)),
        ],
        out_specs=pl.BlockSpec((1, n, fout + 1), lambda h: (h, 0, 0)),
        out_shape=jax.ShapeDtypeStruct((heads, n, fout + 1), jnp.float32),
    )(x, w)


def _attn_common(hall_ref, asrc_ref, adst_ref, out_ref, bias_ref, blk):
    i = pl.program_id(0)
    h = pl.program_id(1)
    f = hall_ref.shape[2] - 1
    hh = hall_ref[h]
    ed = jax.lax.dot_general(adst_ref[h], hh, (((1,), (1,)), ((), ())),
                             preferred_element_type=jnp.float32)
    half = blk // 2
    for k in range(2):
        hh_blk = hall_ref[h, pl.ds(i * blk + k * half, half), :]
        es = jax.lax.dot_general(hh_blk, asrc_ref[h], (((1,), (1,)), ((), ())),
                                 preferred_element_type=jnp.float32)
        e = es + ed
        e = jnp.maximum(e, 0.2 * e) + bias_ref[k * half:(k + 1) * half, :]
        m = jnp.max(e, axis=1, keepdims=True)
        p = jnp.exp(e - m)
        r = jax.lax.dot_general(p, hh, (((1,), (0,)), ((), ())),
                                preferred_element_type=jnp.float32)
        o = r[:, :f] / r[:, f:]
        out_ref[0, k * half:(k + 1) * half, :] = jnp.where(
            o > 0, o, jnp.exp(o) - 1.0)


def _attn1_body(adj_ref, hall_ref, asrc_ref, adst_ref, out_ref, mask_ref,
                bias_ref, *, blk):
    i = pl.program_id(0)
    h = pl.program_id(1)
    n = adj_ref.shape[1]

    @pl.when(h == 0)
    def _():
        rows = jax.lax.broadcasted_iota(jnp.int32, (blk, n), 0) + i * blk
        cols = jax.lax.broadcasted_iota(jnp.int32, (blk, n), 1)
        msk = (adj_ref[...] > 0.99) | (rows == cols)
        mask_ref[...] = msk.astype(jnp.int8)
        bias_ref[...] = msk.astype(jnp.float32) * 1e9 - 1e9

    _attn_common(hall_ref, asrc_ref, adst_ref, out_ref, bias_ref, blk)


def _attn2_body(mask_ref, hall_ref, asrc_ref, adst_ref, out_ref,
                bias_ref, *, blk):
    h = pl.program_id(1)

    @pl.when(h == 0)
    def _():
        bias_ref[...] = mask_ref[...].astype(jnp.float32) * 1e9 - 1e9

    _attn_common(hall_ref, asrc_ref, adst_ref, out_ref, bias_ref, blk)


def _pad_a(a):
    heads, f = a.shape
    return jnp.concatenate([a, jnp.zeros((heads, 1), a.dtype)],
                           axis=1).reshape(heads, 1, f + 1)


def _attn1(adj, hall, a_src, a_dst, blk):
    heads, n, f1 = hall.shape
    f = f1 - 1
    return pl.pallas_call(
        functools.partial(_attn1_body, blk=blk),
        grid=(n // blk, heads),
        in_specs=[
            pl.BlockSpec((blk, n), lambda i, h: (i, 0)),
            pl.BlockSpec((heads, n, f1), lambda i, h: (0, 0, 0)),
            pl.BlockSpec((heads, 1, f1), lambda i, h: (0, 0, 0)),
            pl.BlockSpec((heads, 1, f1), lambda i, h: (0, 0, 0)),
        ],
        out_specs=[
            pl.BlockSpec((1, blk, f), lambda i, h: (h, i, 0)),
            pl.BlockSpec((blk, n), lambda i, h: (i, 0)),
        ],
        out_shape=[
            jax.ShapeDtypeStruct((heads, n, f), jnp.float32),
            jax.ShapeDtypeStruct((n, n), jnp.int8),
        ],
        scratch_shapes=[pltpu.VMEM((blk, n), jnp.float32)],
    )(adj, hall, _pad_a(a_src), _pad_a(a_dst))


def _attn2(mask, hall, a_src, a_dst, blk):
    heads, n, f1 = hall.shape
    f = f1 - 1
    return pl.pallas_call(
        functools.partial(_attn2_body, blk=blk),
        grid=(n // blk, heads),
        in_specs=[
            pl.BlockSpec((blk, n), lambda i, h: (i, 0)),
            pl.BlockSpec((heads, n, f1), lambda i, h: (0, 0, 0)),
            pl.BlockSpec((heads, 1, f1), lambda i, h: (0, 0, 0)),
            pl.BlockSpec((heads, 1, f1), lambda i, h: (0, 0, 0)),
        ],
        out_specs=pl.BlockSpec((1, blk, f), lambda i, h: (h, i, 0)),
        out_shape=jax.ShapeDtypeStruct((heads, n, f), jnp.float32),
        scratch_shapes=[pltpu.VMEM((blk, n), jnp.float32)],
    )(mask, hall, _pad_a(a_src), _pad_a(a_dst))


def _head_body(h2_ref, wd_ref, bd_ref, out_ref):
    s = jnp.sum(h2_ref[...], axis=0, keepdims=True)
    ss = jnp.sum(s * s, axis=1, keepdims=True)
    norm = jnp.maximum(jnp.sqrt(ss), 1e-12)
    s = s / norm
    out_ref[...] = jax.lax.dot_general(
        s, wd_ref[...], (((1,), (0,)), ((), ())),
        preferred_element_type=jnp.float32) + bd_ref[...]


def _head(h2, wd, bd):
    n, d = h2.shape
    return pl.pallas_call(
        _head_body,
        grid=(1,),
        in_specs=[
            pl.BlockSpec((n, d), lambda i: (0, 0)),
            pl.BlockSpec((d, 1), lambda i: (0, 0)),
            pl.BlockSpec((1, 1), lambda i: (0, 0)),
        ],
        out_specs=pl.BlockSpec((1, 1), lambda i: (0, 0)),
        out_shape=jax.ShapeDtypeStruct((1, 1), jnp.float32),
    )(h2, wd, bd)


def kernel(x, adj, W1, a1_src, a1_dst, W2, a2_src, a2_dst, Wd, bd):
    n = x.shape[0]
    blk = 512 if n % 512 == 0 else n

    h1p = _proj(x, W1)
    h1, mask = _attn1(adj, h1p, a1_src, a1_dst, blk)
    h1 = jnp.transpose(h1, (1, 0, 2)).reshape(n, -1)
    h2p = _proj(h1, W2)
    h2 = _attn2(mask, h2p, a2_src, a2_dst, blk)
    h2 = jnp.transpose(h2, (1, 0, 2)).reshape(n, -1)
    out = _head(h2, Wd, bd.reshape(1, 1))
    return out.reshape(1)
